# baseline (speedup 1.0000x reference)
import functools

import jax
import jax.numpy as jnp
from jax import lax
from jax.experimental import pallas as pl
from jax.experimental.pallas import tpu as pltpu
from jax.experimental.pallas import tpu_sc as plsc

_HID = 64
_BATCH = 4096
_SEQ = 24576
_MARGIN = 1.0

_NC = 2
_NS = 16
_L = 16
_NW = _NC * _NS
_ROWS_W = _SEQ // _NW
_CHUNK = 128
_NCH = _ROWS_W // _CHUNK
_NSEG = _HID // _L


def _permute(v, idx):
  dnums = lax.GatherDimensionNumbers(
      offset_dims=(), collapsed_slice_dims=(0,), start_index_map=(0,))
  return lax.gather(v, idx[:, None], dnums, (1,),
                    mode=lax.GatherScatterMode.PROMISE_IN_BOUNDS)


def _build_sc_kernel():
  mesh = plsc.VectorSubcoreMesh(core_axis_name="c", subcore_axis_name="s")

  @functools.partial(
      pl.kernel,
      mesh=mesh,
      compiler_params=pltpu.CompilerParams(use_tc_tiling_on_sc=False),
      out_type=jax.ShapeDtypeStruct((_SEQ,), jnp.float32),
      scratch_types=[
          pltpu.VMEM((_ROWS_W,), jnp.int32),
          pltpu.VMEM((_ROWS_W,), jnp.int32),
          pltpu.VMEM((_ROWS_W,), jnp.int32),
          pltpu.VMEM((_CHUNK, _HID), jnp.float32),
          pltpu.VMEM((_CHUNK, _HID), jnp.float32),
          pltpu.VMEM((_CHUNK, _HID), jnp.float32),
          pltpu.VMEM((_CHUNK,), jnp.float32),
          pltpu.SemaphoreType.DMA,
      ],
  )
  def sc_kernel(h_hbm, t_hbm, r_hbm, ent_hbm, rel_hbm, out_hbm,
                idx_h, idx_t, idx_r, rows_h, rows_t, rows_r, ssq_v, sem):
    wid = lax.axis_index("s") * _NC + lax.axis_index("c")
    ibase = wid * _ROWS_W
    pltpu.sync_copy(h_hbm.at[pl.ds(ibase, _ROWS_W)], idx_h)
    pltpu.sync_copy(t_hbm.at[pl.ds(ibase, _ROWS_W)], idx_t)
    pltpu.sync_copy(r_hbm.at[pl.ds(ibase, _ROWS_W)], idx_r)
    for j in range(_NCH):
      csl = pl.ds(j * _CHUNK, _CHUNK)
      ch = pltpu.async_copy(ent_hbm.at[idx_h.at[csl]], rows_h, sem)
      ct = pltpu.async_copy(ent_hbm.at[idx_t.at[csl]], rows_t, sem)
      cr = pltpu.async_copy(rel_hbm.at[idx_r.at[csl]], rows_r, sem)
      ch.wait()
      ct.wait()
      cr.wait()

      lane = lax.iota(jnp.int32, _L)
      perms = [jnp.bitwise_xor(lane, d) for d in (8, 4, 2, 1)]

      def group_body(g, carry):
        def row_body(k, sv):
          i = g * _L + k
          acc = None
          for q in range(_NSEG):
            sl = pl.ds(q * _L, _L)
            d = rows_h[i, sl] + rows_r[i, sl] - rows_t[i, sl]
            acc = d * d if acc is None else acc + d * d
          for p in perms:
            acc = acc + _permute(acc, p)
          return jnp.where(lane == k, acc, sv)

        sv = lax.fori_loop(0, _L, row_body, jnp.zeros((_L,), jnp.float32))
        ssq_v[pl.ds(g * _L, _L)] = sv
        return carry

      lax.fori_loop(0, _CHUNK // _L, group_body, 0)

      pltpu.sync_copy(ssq_v, out_hbm.at[pl.ds(wid * _ROWS_W + j * _CHUNK, _CHUNK)])

  return sc_kernel


_sc_gather_ssq = _build_sc_kernel()


def _loss_body(ssq_ref, out_ref):
  score = jnp.sqrt(ssq_ref[...])
  p = score[0:1, :]
  n = jnp.mean(score[1:, :], axis=0, keepdims=True)
  out_ref[0, 0] = jnp.sum(jnp.maximum(0.0, p - n + _MARGIN))


_loss_call = pl.pallas_call(
    _loss_body,
    out_shape=jax.ShapeDtypeStruct((1, 1), jnp.float32),
    out_specs=pl.BlockSpec(memory_space=pltpu.SMEM),
)


def kernel(batch_h, batch_t, batch_r, ent_embeddings, rel_embeddings):
  ssq = _sc_gather_ssq(batch_h, batch_t, batch_r, ent_embeddings,
                       rel_embeddings)
  loss = _loss_call(ssq.reshape(_SEQ // _BATCH, _BATCH))
  return loss[0, 0]
